# Initial kernel scaffold; baseline (speedup 1.0000x reference)
import jax
import jax.numpy as jnp
from jax.experimental import pallas as pl


def kernel(pred_boxes, pred_labels, targets, num_targets_per_batch):
    raise NotImplementedError("write your pallas kernel here")


# speedup vs baseline: 116.7315x; 116.7315x over previous
import functools

import jax
import jax.numpy as jnp
from jax import lax
from jax.experimental import pallas as pl
from jax.experimental.pallas import tpu as pltpu


def _lane_pad(n):
    return ((n + 1 + 127) // 128) * 128


def _body(B, N, T, M, pbT_ref, lab_ref, gt_ref, opred_ref, ogt_ref, cost_ref):
    SEN = M - 1
    f32 = jnp.float32
    i32 = jnp.int32
    INF18 = f32(1e18)

    lane1 = lax.broadcasted_iota(i32, (1, M), 1)

    for b in range(B):
        px0 = pbT_ref[b, pl.ds(0, 1), :]
        py0 = pbT_ref[b, pl.ds(1, 1), :]
        px1 = pbT_ref[b, pl.ds(2, 1), :]
        py1 = pbT_ref[b, pl.ds(3, 1), :]
        lab = lab_ref[pl.ds(b, 1), :]
        valid = (lab == 1) & (lane1 < N)
        gx0 = gt_ref[b, 0, pl.ds(0, T), :]
        gy0 = gt_ref[b, 1, pl.ds(0, T), :]
        gx1 = gt_ref[b, 2, pl.ds(0, T), :]
        gy1 = gt_ref[b, 3, pl.ds(0, T), :]
        areag = (gx1 - gx0) * (gy1 - gy0)
        areap = (px1 - px0) * (py1 - py0)
        ltx = jnp.maximum(gx0, px0)
        lty = jnp.maximum(gy0, py0)
        rbx = jnp.minimum(gx1, px1)
        rby = jnp.minimum(gy1, py1)
        inter = jnp.maximum(rbx - ltx, 0.0) * jnp.maximum(rby - lty, 0.0)
        union = areag + areap - inter
        iou = inter / union
        cltx = jnp.minimum(gx0, px0)
        clty = jnp.minimum(gy0, py0)
        crbx = jnp.maximum(gx1, px1)
        crby = jnp.maximum(gy1, py1)
        areac = jnp.maximum(crbx - cltx, 0.0) * jnp.maximum(crby - clty, 0.0)
        giou = iou - (areac - union) / areac
        cost_ref[b] = jnp.where(valid, -giou, jnp.inf)

    lane = lax.broadcasted_iota(i32, (B, M), 1)
    lane128 = lax.broadcasted_iota(i32, (B, 128), 1)

    sub_bm = lax.broadcasted_iota(i32, (B, M), 0)
    sub_b128 = lax.broadcasted_iota(i32, (B, 128), 0)
    sub_b1 = lax.broadcasted_iota(i32, (B, 1), 0)
    zi_bm = jnp.where(sub_bm < 0, 1, 0)
    zf_bm = jnp.where(sub_bm < 0, 1.0, 0.0)
    zi_b128 = jnp.where(sub_b128 < 0, 1, 0)
    zf_b128 = jnp.where(sub_b128 < 0, 1.0, 0.0)
    zi_b1 = jnp.where(sub_b1 < 0, 1, 0)

    u0 = zf_b128
    v0 = zf_bm
    p0 = zi_bm
    way0 = zi_bm + SEN

    def row_step(i, carry):
        u, v, p, way = carry
        p = jnp.where(lane == SEN, i + 1, p)
        minv = zf_bm + INF18
        used = zi_bm
        rowmask = zi_b128
        j0 = zi_b1 + SEN
        act = zi_b1 + 1

        def dcond(st):
            return jnp.max(st[0]) != 0

        def dbody(st):
            act, j0, minv, used, u, v, way, rowmask = st
            sel_j0 = jnp.where(lane == j0, 1, 0)
            used = jnp.maximum(used, act * sel_j0)
            i0 = jnp.sum(sel_j0 * p, axis=1, keepdims=True)
            u_i0 = jnp.sum(jnp.where(lane128 == i0, u, 0.0), axis=1,
                           keepdims=True)
            rowmask = jnp.maximum(
                rowmask, act * jnp.where(lane128 == i0, 1, 0))
            rows = []
            for b in range(B):
                s = jnp.maximum(i0[b, 0] - 1, 0)
                rows.append(cost_ref[b, pl.ds(s, 1), :])
            crow = jnp.concatenate(rows, axis=0)
            cur = crow - u_i0 - v
            freej = 1 - used
            better = act * freej * jnp.where(cur < minv, 1, 0)
            minv = jnp.where(better == 1, cur, minv)
            way = jnp.where(better == 1, j0, way)
            cand = freej * jnp.where(lane < N, 1, 0)
            masked = jnp.where(cand == 1, minv, jnp.inf)
            j1 = jnp.argmin(masked, axis=1, keepdims=True).astype(i32)
            delta = jnp.min(masked, axis=1, keepdims=True)
            u = jnp.where(act * rowmask == 1, u + delta, u)
            v = jnp.where(act * used == 1, v - delta, v)
            minv = jnp.where(act * freej == 1, minv - delta, minv)
            pj1 = jnp.sum(jnp.where(lane == j1, p, 0), axis=1, keepdims=True)
            j0 = jnp.where(act == 1, j1, j0)
            act = act * jnp.where(pj1 != 0, 1, 0)
            return (act, j0, minv, used, u, v, way, rowmask)

        st = lax.while_loop(
            dcond, dbody, (act, j0, minv, used, u, v, way, rowmask))
        _, j0, minv, used, u, v, way, rowmask = st

        def acond(st):
            return jnp.max(st[0]) != 0

        def abody(st):
            aact, j0, p = st
            sel_j0 = jnp.where(lane == j0, 1, 0)
            j1 = jnp.sum(sel_j0 * way, axis=1, keepdims=True)
            pj1 = jnp.sum(jnp.where(lane == j1, p, 0), axis=1, keepdims=True)
            p = jnp.where(aact * sel_j0 == 1, pj1, p)
            j0 = jnp.where(aact == 1, j1, j0)
            aact = aact * jnp.where(j0 != SEN, 1, 0)
            return (aact, j0, p)

        aact = jnp.where(j0 != SEN, 1, 0)
        _, _, p = lax.while_loop(acond, abody, (aact, j0, p))
        return (u, v, p, way)

    u, v, p, way = lax.fori_loop(0, T, row_step, (u0, v0, p0, way0))

    matched = (p != 0) & (lane < N)
    x = matched.astype(i32)
    s = 1
    while s < M:
        x = x + jnp.concatenate(
            [jnp.zeros((B, s), i32), x[:, :M - s]], axis=1)
        s *= 2
    pos = x - 1

    ksub = lax.broadcasted_iota(i32, (128, M), 0)
    jidx = lax.broadcasted_iota(i32, (128, M), 1)
    for b in range(B):
        mrow = matched[b:b + 1, :]
        oh = mrow & (pos[b:b + 1, :] == ksub)
        opred_ref[b] = jnp.sum(jnp.where(oh, jidx, 0), axis=1, keepdims=True)
        ogt_ref[b] = jnp.sum(jnp.where(oh, p[b:b + 1, :] - 1, 0),
                             axis=1, keepdims=True)


@functools.lru_cache(maxsize=None)
def _build(B, N, T):
    M = _lane_pad(N)
    body = functools.partial(_body, B, N, T, M)
    call = pl.pallas_call(
        body,
        out_shape=(
            jax.ShapeDtypeStruct((B, 128, 1), jnp.int32),
            jax.ShapeDtypeStruct((B, 128, 1), jnp.int32),
        ),
        scratch_shapes=[pltpu.VMEM((B, T, M), jnp.float32)],
    )

    def run(pred_boxes, pred_labels, targets):
        pbT = jnp.moveaxis(pred_boxes, 2, 1)
        pbT = jnp.pad(pbT, ((0, 0), (0, 0), (0, M - N)))
        lab = jnp.pad(pred_labels.astype(jnp.int32), ((0, 0), (0, M - N)))
        gtT = jnp.moveaxis(targets.reshape(B, T, 5)[:, :, 1:5], 2, 1)
        gtT = jnp.pad(gtT, ((0, 0), (0, 0), (0, 128 - T)))[..., None]
        opred, ogt = call(pbT, lab, gtT)
        return (opred[:, :T, 0].astype(jnp.int64),
                ogt[:, :T, 0].astype(jnp.int64))

    return run


def kernel(pred_boxes, pred_labels, targets, num_targets_per_batch):
    B, N, _ = pred_boxes.shape
    T = targets.shape[0] // B
    return _build(B, N, T)(pred_boxes, pred_labels, targets)


# speedup vs baseline: 128.3599x; 1.0996x over previous
import functools

import jax
import jax.numpy as jnp
from jax import lax
from jax.experimental import pallas as pl
from jax.experimental.pallas import tpu as pltpu


def _lane_pad(n):
    return ((n + 1 + 127) // 128) * 128


def _body(B, N, T, M, pbT_ref, lab_ref, gt_ref, opred_ref, ogt_ref, cost_ref):
    SEN = M - 1
    f32 = jnp.float32
    i32 = jnp.int32
    INF18 = f32(1e18)

    lane1 = lax.broadcasted_iota(i32, (1, M), 1)

    for b in range(B):
        px0 = pbT_ref[b, pl.ds(0, 1), :]
        py0 = pbT_ref[b, pl.ds(1, 1), :]
        px1 = pbT_ref[b, pl.ds(2, 1), :]
        py1 = pbT_ref[b, pl.ds(3, 1), :]
        lab = lab_ref[pl.ds(b, 1), :]
        valid = (lab == 1) & (lane1 < N)
        gx0 = gt_ref[b, 0, pl.ds(0, T), :]
        gy0 = gt_ref[b, 1, pl.ds(0, T), :]
        gx1 = gt_ref[b, 2, pl.ds(0, T), :]
        gy1 = gt_ref[b, 3, pl.ds(0, T), :]
        areag = (gx1 - gx0) * (gy1 - gy0)
        areap = (px1 - px0) * (py1 - py0)
        ltx = jnp.maximum(gx0, px0)
        lty = jnp.maximum(gy0, py0)
        rbx = jnp.minimum(gx1, px1)
        rby = jnp.minimum(gy1, py1)
        inter = jnp.maximum(rbx - ltx, 0.0) * jnp.maximum(rby - lty, 0.0)
        union = areag + areap - inter
        iou = inter / union
        cltx = jnp.minimum(gx0, px0)
        clty = jnp.minimum(gy0, py0)
        crbx = jnp.maximum(gx1, px1)
        crby = jnp.maximum(gy1, py1)
        areac = jnp.maximum(crbx - cltx, 0.0) * jnp.maximum(crby - clty, 0.0)
        giou = iou - (areac - union) / areac
        cost_ref[b] = jnp.where(valid, -giou, jnp.inf)

    lane = lax.broadcasted_iota(i32, (B, M), 1)
    lane128 = lax.broadcasted_iota(i32, (B, 128), 1)

    sub_bm = lax.broadcasted_iota(i32, (B, M), 0)
    sub_b128 = lax.broadcasted_iota(i32, (B, 128), 0)
    sub_b1 = lax.broadcasted_iota(i32, (B, 1), 0)
    zi_bm = jnp.where(sub_bm < 0, 1, 0)
    zf_bm = jnp.where(sub_bm < 0, 1.0, 0.0)
    zi_b128 = jnp.where(sub_b128 < 0, 1, 0)
    zf_b128 = jnp.where(sub_b128 < 0, 1.0, 0.0)
    zi_b1 = jnp.where(sub_b1 < 0, 1, 0)

    u0 = zf_b128
    v0 = zf_bm
    p0 = zi_bm
    mcol0 = zi_b128
    way0 = zi_bm + SEN
    validlane = jnp.where(lane < N, 1, 0)

    def row_step(i, carry):
        u, v, mcol, p, way = carry
        p = jnp.where(lane == SEN, i + 1, p)
        mcol = jnp.where(lane128 == i + 1, SEN, mcol)
        minv = zf_bm + INF18
        used = zi_bm
        rowmask = zi_b128
        j0 = zi_b1 + SEN
        i0 = zi_b1 + (i + 1)
        act = zi_b1 + 1

        def dcond(st):
            return jnp.max(st[0]) != 0

        def dbody(st):
            act, j0, i0, minv, used, u, v, way, rowmask = st
            sel_j0 = jnp.where(lane == j0, 1, 0)
            used = jnp.maximum(used, act * sel_j0)
            u_i0 = jnp.sum(jnp.where(lane128 == i0, u, 0.0), axis=1,
                           keepdims=True)
            rowmask = jnp.maximum(
                rowmask, act * jnp.where(lane128 == i0, 1, 0))
            rows = []
            for b in range(B):
                s = jnp.maximum(i0[b, 0] - 1, 0)
                rows.append(cost_ref[b, pl.ds(s, 1), :])
            crow = jnp.concatenate(rows, axis=0)
            cur = crow - u_i0 - v
            freej = 1 - used
            better = act * freej * jnp.where(cur < minv, 1, 0)
            minv = jnp.where(better == 1, cur, minv)
            way = jnp.where(better == 1, j0, way)
            masked = jnp.where(freej * validlane == 1, minv, jnp.inf)
            j1 = jnp.argmin(masked, axis=1, keepdims=True).astype(i32)
            delta = jnp.min(masked, axis=1, keepdims=True)
            u = jnp.where(act * rowmask == 1, u + delta, u)
            v = jnp.where(act * used == 1, v - delta, v)
            minv = jnp.where(act * freej == 1, minv - delta, minv)
            pj1 = jnp.sum(jnp.where(mcol == j1, lane128, 0), axis=1,
                          keepdims=True)
            j0 = jnp.where(act == 1, j1, j0)
            i0 = jnp.where(act == 1, pj1, i0)
            act = act * jnp.where(pj1 != 0, 1, 0)
            return (act, j0, i0, minv, used, u, v, way, rowmask)

        st = lax.while_loop(
            dcond, dbody, (act, j0, i0, minv, used, u, v, way, rowmask))
        _, j0, _, minv, used, u, v, way, rowmask = st

        def acond(st):
            return jnp.max(st[0]) != 0

        def abody(st):
            aact, j0, mcol, p = st
            sel_j0 = jnp.where(lane == j0, 1, 0)
            j1 = jnp.sum(sel_j0 * way, axis=1, keepdims=True)
            pj1 = jnp.sum(jnp.where(mcol == j1, lane128, 0), axis=1,
                          keepdims=True)
            p = jnp.where(aact * sel_j0 == 1, pj1, p)
            mcol = jnp.where((lane128 == pj1) * aact == 1, j0, mcol)
            j0 = jnp.where(aact == 1, j1, j0)
            aact = aact * jnp.where(j0 != SEN, 1, 0)
            return (aact, j0, mcol, p)

        aact = jnp.where(j0 != SEN, 1, 0)
        _, _, mcol, p = lax.while_loop(acond, abody, (aact, j0, mcol, p))
        return (u, v, mcol, p, way)

    u, v, mcol, p, way = lax.fori_loop(
        0, T, row_step, (u0, v0, mcol0, p0, way0))

    matched = (p != 0) & (lane < N)
    x = matched.astype(i32)
    s = 1
    while s < M:
        x = x + jnp.concatenate(
            [jnp.zeros((B, s), i32), x[:, :M - s]], axis=1)
        s *= 2
    pos = x - 1

    ksub = lax.broadcasted_iota(i32, (128, M), 0)
    jidx = lax.broadcasted_iota(i32, (128, M), 1)
    for b in range(B):
        mrow = matched[b:b + 1, :]
        oh = mrow & (pos[b:b + 1, :] == ksub)
        opred_ref[b] = jnp.sum(jnp.where(oh, jidx, 0), axis=1, keepdims=True)
        ogt_ref[b] = jnp.sum(jnp.where(oh, p[b:b + 1, :] - 1, 0),
                             axis=1, keepdims=True)


@functools.lru_cache(maxsize=None)
def _build(B, N, T):
    M = _lane_pad(N)
    body = functools.partial(_body, B, N, T, M)
    call = pl.pallas_call(
        body,
        out_shape=(
            jax.ShapeDtypeStruct((B, 128, 1), jnp.int32),
            jax.ShapeDtypeStruct((B, 128, 1), jnp.int32),
        ),
        scratch_shapes=[pltpu.VMEM((B, T, M), jnp.float32)],
    )

    def run(pred_boxes, pred_labels, targets):
        pbT = jnp.moveaxis(pred_boxes, 2, 1)
        pbT = jnp.pad(pbT, ((0, 0), (0, 0), (0, M - N)))
        lab = jnp.pad(pred_labels.astype(jnp.int32), ((0, 0), (0, M - N)))
        gtT = jnp.moveaxis(targets.reshape(B, T, 5)[:, :, 1:5], 2, 1)
        gtT = jnp.pad(gtT, ((0, 0), (0, 0), (0, 128 - T)))[..., None]
        opred, ogt = call(pbT, lab, gtT)
        return (opred[:, :T, 0].astype(jnp.int64),
                ogt[:, :T, 0].astype(jnp.int64))

    return run


def kernel(pred_boxes, pred_labels, targets, num_targets_per_batch):
    B, N, _ = pred_boxes.shape
    T = targets.shape[0] // B
    return _build(B, N, T)(pred_boxes, pred_labels, targets)


# speedup vs baseline: 141.7824x; 1.1046x over previous
import functools

import jax
import jax.numpy as jnp
from jax import lax
from jax.experimental import pallas as pl
from jax.experimental.pallas import tpu as pltpu


def _lane_pad(n):
    return ((n + 1 + 127) // 128) * 128


def _body(B, N, T, M, pbT_ref, lab_ref, gt_ref, opred_ref, ogt_ref, cost_ref):
    SEN = M - 1
    f32 = jnp.float32
    i32 = jnp.int32
    INF18 = f32(1e18)

    lane1 = lax.broadcasted_iota(i32, (1, M), 1)

    for b in range(B):
        px0 = pbT_ref[b, pl.ds(0, 1), :]
        py0 = pbT_ref[b, pl.ds(1, 1), :]
        px1 = pbT_ref[b, pl.ds(2, 1), :]
        py1 = pbT_ref[b, pl.ds(3, 1), :]
        lab = lab_ref[pl.ds(b, 1), :]
        valid = (lab == 1) & (lane1 < N)
        gx0 = gt_ref[b, 0, pl.ds(0, T), :]
        gy0 = gt_ref[b, 1, pl.ds(0, T), :]
        gx1 = gt_ref[b, 2, pl.ds(0, T), :]
        gy1 = gt_ref[b, 3, pl.ds(0, T), :]
        areag = (gx1 - gx0) * (gy1 - gy0)
        areap = (px1 - px0) * (py1 - py0)
        ltx = jnp.maximum(gx0, px0)
        lty = jnp.maximum(gy0, py0)
        rbx = jnp.minimum(gx1, px1)
        rby = jnp.minimum(gy1, py1)
        inter = jnp.maximum(rbx - ltx, 0.0) * jnp.maximum(rby - lty, 0.0)
        union = areag + areap - inter
        iou = inter / union
        cltx = jnp.minimum(gx0, px0)
        clty = jnp.minimum(gy0, py0)
        crbx = jnp.maximum(gx1, px1)
        crby = jnp.maximum(gy1, py1)
        areac = jnp.maximum(crbx - cltx, 0.0) * jnp.maximum(crby - clty, 0.0)
        giou = iou - (areac - union) / areac
        cost_ref[b] = jnp.where(valid, -giou, jnp.inf)

    lane = lax.broadcasted_iota(i32, (B, M), 1)
    lane128 = lax.broadcasted_iota(i32, (B, 128), 1)

    sub_bm = lax.broadcasted_iota(i32, (B, M), 0)
    sub_b128 = lax.broadcasted_iota(i32, (B, 128), 0)
    sub_b1 = lax.broadcasted_iota(i32, (B, 1), 0)
    zi_bm = jnp.where(sub_bm < 0, 1, 0)
    zf_bm = jnp.where(sub_bm < 0, 1.0, 0.0)
    zi_b128 = jnp.where(sub_b128 < 0, 1, 0)
    zf_b128 = jnp.where(sub_b128 < 0, 1.0, 0.0)
    zi_b1 = jnp.where(sub_b1 < 0, 1, 0)

    u0 = zf_b128
    v0 = zf_bm
    p0 = zi_bm
    mcol0 = zi_b128
    way0 = zi_bm + SEN
    validlane = jnp.where(lane < N, 1, 0)

    def row_step(i, carry):
        u, v, mcol, p, way = carry
        p = jnp.where(lane == SEN, i + 1, p)
        mcol = jnp.where(lane128 == i + 1, SEN, mcol)
        D = jnp.where(validlane == 1, zf_bm + INF18, jnp.inf)
        rowmask = zi_b128
        j0 = zi_b1 + SEN
        i0 = zi_b1 + (i + 1)
        act = zi_b1 + 1

        def dcond(st):
            return jnp.max(st[0]) != 0

        def dbody(st):
            act, j0, i0, D, u, v, way, rowmask = st
            u_i0 = jnp.sum(jnp.where(lane128 == i0, u, 0.0), axis=1,
                           keepdims=True)
            rowmask = jnp.maximum(
                rowmask, act * jnp.where(lane128 == i0, 1, 0))
            rows = []
            for b in range(B):
                s = jnp.maximum(i0[b, 0] - 1, 0)
                rows.append(cost_ref[b, pl.ds(s, 1), :])
            crow = jnp.concatenate(rows, axis=0)
            cur = crow - u_i0 - v
            dfin = jnp.where(D < jnp.inf, 1, 0)
            better = act * dfin * jnp.where(cur < D, 1, 0)
            D = jnp.where(better == 1, cur, D)
            way = jnp.where(better == 1, j0, way)
            j1 = jnp.argmin(D, axis=1, keepdims=True).astype(i32)
            delta = jnp.min(D, axis=1, keepdims=True)
            u = jnp.where(act * rowmask == 1, u + delta, u)
            dfin = jnp.where(D < jnp.inf, 1, 0)
            v = jnp.where(act * (1 - dfin) == 1, v - delta, v)
            D = jnp.where(act * dfin == 1, D - delta, D)
            D = jnp.where((lane == j1) & (act == 1), jnp.inf, D)
            pj1 = jnp.sum(jnp.where(mcol == j1, lane128, 0), axis=1,
                          keepdims=True)
            j0 = jnp.where(act == 1, j1, j0)
            i0 = jnp.where(act == 1, pj1, i0)
            act = act * jnp.where(pj1 != 0, 1, 0)
            return (act, j0, i0, D, u, v, way, rowmask)

        st = lax.while_loop(
            dcond, dbody, (act, j0, i0, D, u, v, way, rowmask))
        _, j0, _, D, u, v, way, rowmask = st

        def acond(st):
            return jnp.max(st[0]) != 0

        def abody(st):
            aact, j0, mcol, p = st
            sel_j0 = jnp.where(lane == j0, 1, 0)
            j1 = jnp.sum(sel_j0 * way, axis=1, keepdims=True)
            pj1 = jnp.sum(jnp.where(mcol == j1, lane128, 0), axis=1,
                          keepdims=True)
            p = jnp.where(aact * sel_j0 == 1, pj1, p)
            mcol = jnp.where((lane128 == pj1) * aact == 1, j0, mcol)
            j0 = jnp.where(aact == 1, j1, j0)
            aact = aact * jnp.where(j0 != SEN, 1, 0)
            return (aact, j0, mcol, p)

        aact = jnp.where(j0 != SEN, 1, 0)
        _, _, mcol, p = lax.while_loop(acond, abody, (aact, j0, mcol, p))
        return (u, v, mcol, p, way)

    u, v, mcol, p, way = lax.fori_loop(
        0, T, row_step, (u0, v0, mcol0, p0, way0))

    matched = (p != 0) & (lane < N)
    x = matched.astype(i32)
    s = 1
    while s < M:
        x = x + jnp.concatenate(
            [jnp.zeros((B, s), i32), x[:, :M - s]], axis=1)
        s *= 2
    pos = x - 1

    packed = jnp.where(matched, lane * 8192 + (p - 1), 0)
    ksub = lax.broadcasted_iota(i32, (128, M), 0)
    for b in range(B):
        oh = matched[b:b + 1, :] & (pos[b:b + 1, :] == ksub)
        c = jnp.sum(jnp.where(oh, packed[b:b + 1, :], 0),
                    axis=1, keepdims=True)
        opred_ref[b] = c >> 13
        ogt_ref[b] = c & 8191


@functools.lru_cache(maxsize=None)
def _build(B, N, T):
    M = _lane_pad(N)
    body = functools.partial(_body, B, N, T, M)
    call = pl.pallas_call(
        body,
        out_shape=(
            jax.ShapeDtypeStruct((B, 128, 1), jnp.int32),
            jax.ShapeDtypeStruct((B, 128, 1), jnp.int32),
        ),
        scratch_shapes=[pltpu.VMEM((B, T, M), jnp.float32)],
    )

    def run(pred_boxes, pred_labels, targets):
        pbT = jnp.moveaxis(pred_boxes, 2, 1)
        pbT = jnp.pad(pbT, ((0, 0), (0, 0), (0, M - N)))
        lab = jnp.pad(pred_labels.astype(jnp.int32), ((0, 0), (0, M - N)))
        gtT = jnp.moveaxis(targets.reshape(B, T, 5)[:, :, 1:5], 2, 1)
        gtT = jnp.pad(gtT, ((0, 0), (0, 0), (0, 128 - T)))[..., None]
        opred, ogt = call(pbT, lab, gtT)
        return (opred[:, :T, 0].astype(jnp.int64),
                ogt[:, :T, 0].astype(jnp.int64))

    return run


def kernel(pred_boxes, pred_labels, targets, num_targets_per_batch):
    B, N, _ = pred_boxes.shape
    T = targets.shape[0] // B
    return _build(B, N, T)(pred_boxes, pred_labels, targets)


# speedup vs baseline: 176.1910x; 1.2427x over previous
import functools

import jax
import jax.numpy as jnp
from jax import lax
from jax.experimental import pallas as pl
from jax.experimental.pallas import tpu as pltpu


def _lane_pad(n):
    return ((n + 1 + 127) // 128) * 128


def _body(B, N, T, M, TP, pbT_ref, lab_ref, gt_ref, opred_ref, ogt_ref, cost_ref):
    SEN = M - 1
    f32 = jnp.float32
    i32 = jnp.int32
    INF18 = f32(1e18)

    lane1 = lax.broadcasted_iota(i32, (1, M), 1)

    for b in range(B):
        px0 = pbT_ref[b, pl.ds(0, 1), :]
        py0 = pbT_ref[b, pl.ds(1, 1), :]
        px1 = pbT_ref[b, pl.ds(2, 1), :]
        py1 = pbT_ref[b, pl.ds(3, 1), :]
        lab = lab_ref[pl.ds(b, 1), :]
        valid = (lab == 1) & (lane1 < N)
        gx0 = gt_ref[b, 0, pl.ds(0, TP), :]
        gy0 = gt_ref[b, 1, pl.ds(0, TP), :]
        gx1 = gt_ref[b, 2, pl.ds(0, TP), :]
        gy1 = gt_ref[b, 3, pl.ds(0, TP), :]
        areag = (gx1 - gx0) * (gy1 - gy0)
        areap = (px1 - px0) * (py1 - py0)
        ltx = jnp.maximum(gx0, px0)
        lty = jnp.maximum(gy0, py0)
        rbx = jnp.minimum(gx1, px1)
        rby = jnp.minimum(gy1, py1)
        inter = jnp.maximum(rbx - ltx, 0.0) * jnp.maximum(rby - lty, 0.0)
        union = areag + areap - inter
        iou = inter / union
        cltx = jnp.minimum(gx0, px0)
        clty = jnp.minimum(gy0, py0)
        crbx = jnp.maximum(gx1, px1)
        crby = jnp.maximum(gy1, py1)
        areac = jnp.maximum(crbx - cltx, 0.0) * jnp.maximum(crby - clty, 0.0)
        giou = iou - (areac - union) / areac
        cost_ref[b] = jnp.where(valid, -giou, jnp.inf)

    lane = lax.broadcasted_iota(i32, (B, M), 1)
    lane128 = lax.broadcasted_iota(i32, (B, 128), 1)

    sub_bm = lax.broadcasted_iota(i32, (B, M), 0)
    sub_b128 = lax.broadcasted_iota(i32, (B, 128), 0)
    sub_b1 = lax.broadcasted_iota(i32, (B, 1), 0)
    zi_bm = jnp.where(sub_bm < 0, 1, 0)
    zf_bm = jnp.where(sub_bm < 0, 1.0, 0.0)
    zi_b128 = jnp.where(sub_b128 < 0, 1, 0)
    zf_b128 = jnp.where(sub_b128 < 0, 1.0, 0.0)
    zi_b1 = jnp.where(sub_b1 < 0, 1, 0)

    u0 = zf_b128
    v0 = zf_bm
    p0 = zi_bm
    mcol0 = zi_b128 - 1
    way0 = zi_bm + SEN
    validlane = jnp.where(lane < N, 1, 0)

    def row_step(i, carry):
        u, v, mcol, p, way = carry
        p = jnp.where(lane == SEN, i + 1, p)
        mcol = jnp.where(lane128 == i + 1, SEN, mcol)
        D = jnp.where(validlane == 1, zf_bm + INF18, jnp.inf)
        rowmask = zi_b128
        j0 = zi_b1 + SEN
        i0 = zi_b1 + (i + 1)
        act = zi_b1 + 1

        def dcond(st):
            return jnp.max(st[0]) != 0

        def dbody(st):
            act, j0, i0, D, u, v, way, rowmask = st
            u_i0 = jnp.sum(jnp.where(lane128 == i0, u, 0.0), axis=1,
                           keepdims=True)
            rowmask = jnp.maximum(
                rowmask, act * jnp.where(lane128 == i0, 1, 0))
            rows = []
            for b in range(B):
                s = jnp.maximum(i0[b, 0] - 1, 0)
                rows.append(cost_ref[b, pl.ds(s, 1), :])
            crow = jnp.concatenate(rows, axis=0)
            cur = crow - u_i0 - v
            dfin = jnp.where(D < jnp.inf, 1, 0)
            better = act * dfin * jnp.where(cur < D, 1, 0)
            D = jnp.where(better == 1, cur, D)
            way = jnp.where(better == 1, j0, way)
            j1 = jnp.argmin(D, axis=1, keepdims=True).astype(i32)
            delta = jnp.min(D, axis=1, keepdims=True)
            u = jnp.where(act * rowmask == 1, u + delta, u)
            v = jnp.where(act * (1 - dfin) == 1, v - delta, v)
            D = jnp.where(act * dfin == 1, D - delta, D)
            D = jnp.where((lane == j1) & (act == 1), jnp.inf, D)
            pj1 = jnp.sum(jnp.where(mcol == j1, lane128, 0), axis=1,
                          keepdims=True)
            j0 = jnp.where(act == 1, j1, j0)
            i0 = jnp.where(act == 1, pj1, i0)
            act = act * jnp.where(pj1 != 0, 1, 0)
            return (act, j0, i0, D, u, v, way, rowmask)

        st = lax.while_loop(
            dcond, dbody, (act, j0, i0, D, u, v, way, rowmask))
        _, j0, _, D, u, v, way, rowmask = st

        def acond(st):
            return jnp.max(st[0]) != 0

        def abody(st):
            aact, j0, mcol, p = st
            sel_j0 = jnp.where(lane == j0, 1, 0)
            j1 = jnp.sum(sel_j0 * way, axis=1, keepdims=True)
            pj1 = jnp.sum(jnp.where(mcol == j1, lane128, 0), axis=1,
                          keepdims=True)
            p = jnp.where(aact * sel_j0 == 1, pj1, p)
            mcol = jnp.where((lane128 == pj1) * aact == 1, j0, mcol)
            j0 = jnp.where(aact == 1, j1, j0)
            aact = aact * jnp.where(j0 != SEN, 1, 0)
            return (aact, j0, mcol, p)

        aact = jnp.where(j0 != SEN, 1, 0)
        _, _, mcol, p = lax.while_loop(acond, abody, (aact, j0, mcol, p))
        return (u, v, mcol, p, way)

    u, v, mcol, p, way = lax.fori_loop(
        0, T, row_step, (u0, v0, mcol0, p0, way0))

    matched = (p != 0) & (lane < N)
    x = matched.astype(i32)
    s = 1
    while s < M:
        x = x + jnp.concatenate(
            [jnp.zeros((B, s), i32), x[:, :M - s]], axis=1)
        s *= 2
    pos = x - 1

    packed = jnp.where(matched, lane * 8192 + (p - 1), 0)
    d = jnp.where(matched, lane - pos, 0)
    arr = packed
    s = 1
    while s < M:
        zs = jnp.zeros((B, s), i32)
        arr_sh = jnp.concatenate([arr[:, s:], zs], axis=1)
        d_sh = jnp.concatenate([d[:, s:], zs], axis=1)
        tk = jnp.where((d_sh & s) != 0, 1, 0)
        arr = jnp.where(tk == 1, arr_sh, arr)
        d = jnp.where(tk == 1, d_sh - s, d)
        s *= 2
    c = arr[:, :128]
    opred_ref[...] = c >> 13
    ogt_ref[...] = c & 8191


@functools.lru_cache(maxsize=None)
def _build(B, N, T):
    M = _lane_pad(N)
    TP = ((T + 7) // 8) * 8
    body = functools.partial(_body, B, N, T, M, TP)
    call = pl.pallas_call(
        body,
        out_shape=(
            jax.ShapeDtypeStruct((B, 128), jnp.int32),
            jax.ShapeDtypeStruct((B, 128), jnp.int32),
        ),
        scratch_shapes=[pltpu.VMEM((B, TP, M), jnp.float32)],
    )

    def run(pred_boxes, pred_labels, targets):
        pbT = jnp.moveaxis(pred_boxes, 2, 1)
        pbT = jnp.pad(pbT, ((0, 0), (0, 0), (0, M - N)))
        lab = jnp.pad(pred_labels.astype(jnp.int32), ((0, 0), (0, M - N)))
        gtT = jnp.moveaxis(targets.reshape(B, T, 5)[:, :, 1:5], 2, 1)
        gtT = jnp.pad(gtT, ((0, 0), (0, 0), (0, 128 - T)))[..., None]
        opred, ogt = call(pbT, lab, gtT)
        return (opred[:, :T].astype(jnp.int64),
                ogt[:, :T].astype(jnp.int64))

    return run


def kernel(pred_boxes, pred_labels, targets, num_targets_per_batch):
    B, N, _ = pred_boxes.shape
    T = targets.shape[0] // B
    return _build(B, N, T)(pred_boxes, pred_labels, targets)
